# Initial kernel scaffold; baseline (speedup 1.0000x reference)
import jax
import jax.numpy as jnp
from jax.experimental import pallas as pl


def kernel(node_attrs, node_feats, edge_attrs, edge_feats, W_up, W_lin, W_skip, edge_index):
    raise NotImplementedError("write your pallas kernel here")


# speedup vs baseline: 12.2286x; 12.2286x over previous
import jax
import jax.numpy as jnp
from jax import lax
from jax.experimental import pallas as pl
from jax.experimental.pallas import tpu as pltpu
from jax.experimental.pallas import tpu_sc as plsc

N = 10000
E = 160000
C = 128
NSPH = 4
NEL = 10
AVG_NEIGH = 16.0

NCHUNK = 4
CW = C // NCHUNK
B = 80
NBATCH = E // B
NBATCH_P = 2048
E_P = NBATCH_P * B
BATCH_PER_TILE = NBATCH_P // 16
EDGES_PER_TILE = BATCH_PER_TILE * B
N_P = 10112
NODE_ROWS_PER_TILE = N_P // 16
ZCH = 88


def _up_body(x_ref, w_ref, out_ref):
    out_ref[...] = jnp.dot(x_ref[...], w_ref[...],
                           preferred_element_type=jnp.float32)


def _linear_up(node_feats, w_up):
    blk = 1000
    return pl.pallas_call(
        _up_body,
        grid=(N // blk,),
        in_specs=[
            pl.BlockSpec((blk, C), lambda i: (i, 0)),
            pl.BlockSpec((C, C), lambda i: (0, 0)),
        ],
        out_specs=pl.BlockSpec((blk, C), lambda i: (i, 0)),
        out_shape=jax.ShapeDtypeStruct((N, C), jnp.float32),
    )(node_feats, w_up)


def _eft_body(x_ref, out_ref):
    x = x_ref[...]
    for cc in range(NCHUNK):
        out_ref[cc, :, 0:CW] = x[:, cc * CW:(cc + 1) * CW]
        out_ref[cc, :, CW:2 * CW] = x[:, C + cc * CW:C + (cc + 1) * CW]


def _transpose_ef(edge_feats):
    blk = 640
    return pl.pallas_call(
        _eft_body,
        grid=(E // blk,),
        in_specs=[pl.BlockSpec((blk, 2 * C), lambda i: (i, 0))],
        out_specs=pl.BlockSpec((NCHUNK, blk, 2 * CW), lambda i: (0, i, 0)),
        out_shape=jax.ShapeDtypeStruct((NCHUNK, E, 2 * CW), jnp.float32),
    )(edge_feats)


def _w2_body(wl_ref, ws_ref, out_ref):
    out_ref[0, 0] = jnp.dot(
        wl_ref[0], ws_ref[0, 0], preferred_element_type=jnp.float32
    ) * (1.0 / AVG_NEIGH)


def _combine_weights(w_lin, w_skip):
    return pl.pallas_call(
        _w2_body,
        grid=(NEL, 2),
        in_specs=[
            pl.BlockSpec((1, C, C), lambda e, l: (l, 0, 0)),
            pl.BlockSpec((1, 1, C, C), lambda e, l: (e, l, 0, 0)),
        ],
        out_specs=pl.BlockSpec((1, 1, C, C), lambda e, l: (e, l, 0, 0)),
        out_shape=jax.ShapeDtypeStruct((NEL, 2, C, C), jnp.float32),
    )(w_lin, w_skip)


def _sc_body(h, attrs, eft, send2d, recv2d, msg,
             send_v, recv_v, a_v, ef_v, g_v, stage_v, acc, gsem):
    core = lax.axis_index("c")
    t = lax.axis_index("s")

    limit = jnp.where(t == 15, BATCH_PER_TILE - 48, BATCH_PER_TILE)

    for cc in range(2):
        ccg = 2 * core + cc
        col0 = ccg * CW

        def _zb(i, carry):
            for hh in range(8):
                stage_v[i, pl.ds(hh * 16, 16)] = jnp.zeros((16,), jnp.float32)
            return carry
        lax.fori_loop(0, B, _zb, 0)
        for k in range(7):
            pltpu.sync_copy(
                stage_v, acc.at[pl.ds(t * NODE_ROWS_PER_TILE + k * B, B)])
        pltpu.sync_copy(
            stage_v.at[pl.ds(0, NODE_ROWS_PER_TILE - 7 * B)],
            acc.at[pl.ds(t * NODE_ROWS_PER_TILE + 7 * B,
                         NODE_ROWS_PER_TILE - 7 * B)])
        plsc.subcore_barrier()

        def _batch(b, carry):
            e0 = t * EDGES_PER_TILE + b * B

            @pl.when(b % 32 == 0)
            def _():
                j = b // 32
                pltpu.sync_copy(
                    recv2d.at[pl.ds(t * BATCH_PER_TILE + j * 32, 32)], recv_v)
                pltpu.sync_copy(
                    send2d.at[pl.ds(t * BATCH_PER_TILE + j * 32, 32)], send_v)

            @pl.when(b % 16 == 0)
            def _():
                j = b // 16
                pltpu.sync_copy(attrs.at[pl.ds(t * 320 + j * 40, 40)], a_v)

            bl = b % 32
            pltpu.async_copy(h.at[send_v.at[bl]], g_v, gsem).wait()
            pltpu.sync_copy(eft.at[ccg, pl.ds(e0, B), :], ef_v)

            def _group(grp, c2):
                i_grp = (b % 16) * (B // 4) + grp
                arow = a_v[i_grp >> 3, pl.ds((i_grp & 7) * 16, 16)]
                for k in range(4):
                    e = grp * 4 + k
                    a0 = arow[4 * k + 0]
                    a1 = arow[4 * k + 1]
                    a2 = arow[4 * k + 2]
                    a3 = arow[4 * k + 3]
                    for hh in range(2):
                        sl = pl.ds(hh * 16, 16)
                        g = g_v[e, pl.ds(col0 + hh * 16, 16)]
                        u0 = ef_v[e, sl] * g
                        u1 = ef_v[e, pl.ds(CW + hh * 16, 16)] * g
                        stage_v[e, sl] = a0 * u0
                        stage_v[e, pl.ds(CW + hh * 16, 16)] = a1 * u1
                        stage_v[e, pl.ds(2 * CW + hh * 16, 16)] = a2 * u1
                        stage_v[e, pl.ds(3 * CW + hh * 16, 16)] = a3 * u1
                return c2
            lax.fori_loop(0, B // 4, _group, 0)

            pltpu.sync_copy(stage_v, acc.at[recv_v.at[bl]], add=True)
            return carry
        lax.fori_loop(0, limit, _batch, 0)

        plsc.subcore_barrier()
        pltpu.sync_copy(
            acc.at[pl.ds(t * NODE_ROWS_PER_TILE, NODE_ROWS_PER_TILE)],
            msg.at[ccg, pl.ds(t * NODE_ROWS_PER_TILE, NODE_ROWS_PER_TILE), :])
        plsc.subcore_barrier()


def _message_passing(h, attrs, eft, send2d, recv2d):
    mesh = plsc.VectorSubcoreMesh(core_axis_name="c", subcore_axis_name="s")
    return pl.kernel(
        _sc_body,
        out_type=jax.ShapeDtypeStruct((NCHUNK, N_P, C), jnp.float32),
        mesh=mesh,
        scratch_types=[
            pltpu.VMEM((32, B), jnp.int32),
            pltpu.VMEM((32, B), jnp.int32),
            pltpu.VMEM((40, 128), jnp.float32),
            pltpu.VMEM((B, 2 * CW), jnp.float32),
            pltpu.VMEM((B, C), jnp.float32),
            pltpu.VMEM((B, C), jnp.float32),
            pltpu.VMEM_SHARED((N_P, C), jnp.float32),
            pltpu.SemaphoreType.DMA,
        ],
    )(h, attrs, eft, send2d, recv2d)


def _out_body(msg_ref, na_ref, w2_ref, out_ref):
    na = na_ref[...]
    maxv = jnp.max(na, axis=1, keepdims=True)
    iota = lax.broadcasted_iota(jnp.int32, na.shape, 1)
    elem = jnp.min(jnp.where(na == maxv, iota, NEL), axis=1, keepdims=True)
    for m in range(NSPH):
        l = 0 if m == 0 else 1
        x_m = jnp.concatenate(
            [msg_ref[cc, :, m * CW:(m + 1) * CW] for cc in range(NCHUNK)],
            axis=-1)
        acc = jnp.zeros(x_m.shape, jnp.float32)
        for e in range(NEL):
            sel = (elem == e).astype(jnp.float32)
            acc = acc + sel * jnp.dot(
                x_m, w2_ref[e, l], preferred_element_type=jnp.float32)
        out_ref[:, m, :] = acc


def _skip_mix(msg, node_attrs, w2):
    blk = 1000
    return pl.pallas_call(
        _out_body,
        grid=(N // blk,),
        in_specs=[
            pl.BlockSpec((NCHUNK, blk, C), lambda i: (0, i, 0)),
            pl.BlockSpec((blk, NEL), lambda i: (i, 0)),
            pl.BlockSpec((NEL, 2, C, C), lambda i: (0, 0, 0, 0)),
        ],
        out_specs=pl.BlockSpec((blk, NSPH, C), lambda i: (i, 0, 0)),
        out_shape=jax.ShapeDtypeStruct((N, NSPH, C), jnp.float32),
    )(msg, node_attrs, w2)


def kernel(node_attrs, node_feats, edge_attrs, edge_feats, W_up, W_lin,
           W_skip, edge_index):
    sender = edge_index[1]
    receiver = edge_index[0]
    npad = E_P - E
    sender_p = jnp.concatenate(
        [sender, jnp.zeros((npad,), jnp.int32)])
    receiver_p = jnp.concatenate(
        [receiver,
         N + (jnp.arange(npad, dtype=jnp.int32) % 16)])
    send2d = sender_p.reshape(NBATCH_P, B)
    recv2d = receiver_p.reshape(NBATCH_P, B)
    attrs = jnp.concatenate(
        [edge_attrs.reshape(E * NSPH),
         jnp.zeros((npad * NSPH,), jnp.float32)]).reshape(E_P // 32, 128)

    h = _linear_up(node_feats, W_up)
    eft = _transpose_ef(edge_feats)
    w2 = _combine_weights(W_lin, W_skip)
    msg = _message_passing(h, attrs, eft, send2d, recv2d)
    return _skip_mix(msg, node_attrs, w2)
